# baseline (speedup 1.0000x reference)
import functools

import jax
import jax.numpy as jnp
from jax import lax
from jax.experimental import pallas as pl
from jax.experimental.pallas import tpu as pltpu, tpu_sc as plsc

GRID_N = 256
NUM_Q = 1000000

NC = 2
NS = 16
LANES = 16
NW = NC * NS

CHUNK = 992
CHUNKS_PER_W = 32
Q_PER_W = CHUNK * CHUNKS_PER_W

LAST_LOAD = NUM_Q - CHUNK
STRADDLE_BASE = (NUM_Q // CHUNK) * CHUNK
STRADDLE_OFF = STRADDLE_BASE - LAST_LOAD
STRADDLE_LEN = NUM_Q - STRADDLE_BASE

_OFFS = (0, GRID_N, GRID_N * GRID_N, GRID_N * GRID_N + GRID_N)


def _sc_interp(pairs, x0, x1, x2):
    mesh = plsc.VectorSubcoreMesh(
        core_axis_name="c", subcore_axis_name="s",
        num_cores=NC, num_subcores=NS)

    @functools.partial(
        pl.kernel,
        out_type=jax.ShapeDtypeStruct((NUM_Q,), jnp.float32),
        mesh=mesh,
        scratch_types=dict(
            xb=[pltpu.VMEM((CHUNK,), jnp.float32) for _ in range(3)],
            fb=[[pltpu.VMEM((CHUNK,), jnp.float32) for _ in range(3)]
                for _ in range(2)],
            idx=[[pltpu.VMEM((CHUNK,), jnp.int32) for _ in range(4)]
                 for _ in range(2)],
            cb=[[pltpu.VMEM((CHUNK,), jnp.int32) for _ in range(4)]
                for _ in range(2)],
            ob=pltpu.VMEM((CHUNK,), jnp.float32),
            sem=[pltpu.SemaphoreType.DMA for _ in range(2)],
        ),
    )
    def body(pairs_hbm, x0_hbm, x1_hbm, x2_hbm, out_hbm,
             xb, fb, idx, cb, ob, sem):
        wid = lax.axis_index("s") * NC + lax.axis_index("c")
        wbase = wid * Q_PER_W

        def compute_and_fire(ci, b):
            base = jnp.minimum(wbase + ci * CHUNK, LAST_LOAD)
            pltpu.sync_copy(x0_hbm.at[pl.ds(base, CHUNK)], xb[0])
            pltpu.sync_copy(x1_hbm.at[pl.ds(base, CHUNK)], xb[1])
            pltpu.sync_copy(x2_hbm.at[pl.ds(base, CHUNK)], xb[2])

            def compute(i, _):
                s = pl.ds(i * LANES, LANES)
                xv0 = xb[0][s]
                xv1 = xb[1][s]
                xv2 = xb[2][s]
                i0 = jnp.minimum(xv0.astype(jnp.int32), GRID_N - 2)
                i1 = jnp.minimum(xv1.astype(jnp.int32), GRID_N - 2)
                i2 = jnp.minimum(xv2.astype(jnp.int32), GRID_N - 2)
                fb[b][0][s] = xv0 - i0.astype(jnp.float32)
                fb[b][1][s] = xv1 - i1.astype(jnp.float32)
                fb[b][2][s] = xv2 - i2.astype(jnp.float32)
                flat = i0 * (GRID_N * GRID_N) + i1 * GRID_N + i2
                for k in range(4):
                    idx[b][k][s] = flat + _OFFS[k]
                return 0

            lax.fori_loop(0, CHUNK // LANES, compute, 0)
            for k in range(4):
                pltpu.async_copy(pairs_hbm.at[idx[b][k]], cb[b][k], sem[b])

        def drain_combine_store(ci, b):
            for k in range(4):
                pltpu.make_async_copy(
                    pairs_hbm.at[idx[b][k]], cb[b][k], sem[b]).wait()

            def combine(i, _):
                s = pl.ds(i * LANES, LANES)
                f0 = fb[b][0][s]
                f1 = fb[b][1][s]
                f2 = fb[b][2][s]
                w00 = cb[b][0][s]
                w01 = cb[b][1][s]
                w10 = cb[b][2][s]
                w11 = cb[b][3][s]
                c000 = lax.bitcast_convert_type(w00 << 16, jnp.float32)
                c001 = lax.bitcast_convert_type(w00 & -65536, jnp.float32)
                c010 = lax.bitcast_convert_type(w01 << 16, jnp.float32)
                c011 = lax.bitcast_convert_type(w01 & -65536, jnp.float32)
                c100 = lax.bitcast_convert_type(w10 << 16, jnp.float32)
                c101 = lax.bitcast_convert_type(w10 & -65536, jnp.float32)
                c110 = lax.bitcast_convert_type(w11 << 16, jnp.float32)
                c111 = lax.bitcast_convert_type(w11 & -65536, jnp.float32)
                v00 = c000 + f2 * (c001 - c000)
                v01 = c010 + f2 * (c011 - c010)
                v10 = c100 + f2 * (c101 - c100)
                v11 = c110 + f2 * (c111 - c110)
                v0 = v00 + f1 * (v01 - v00)
                v1 = v10 + f1 * (v11 - v10)
                ob[s] = v0 + f0 * (v1 - v0)
                return 0

            lax.fori_loop(0, CHUNK // LANES, combine, 0)
            base = wbase + ci * CHUNK

            @pl.when(base + CHUNK <= NUM_Q)
            def _():
                pltpu.sync_copy(ob, out_hbm.at[pl.ds(base, CHUNK)])

            @pl.when(base == STRADDLE_BASE)
            def _():
                pltpu.sync_copy(
                    ob.at[pl.ds(STRADDLE_OFF, STRADDLE_LEN)],
                    out_hbm.at[pl.ds(STRADDLE_BASE, STRADDLE_LEN)])

        def live(ci):
            return wbase + ci * CHUNK < NUM_Q

        @pl.when(live(0))
        def _():
            compute_and_fire(0, 0)

        def pair_body(i, _):
            for s in range(2):
                ci = 2 * i + s

                @pl.when(live(ci + 1) & (ci < CHUNKS_PER_W - 1))
                def _():
                    compute_and_fire(ci + 1, 1 - s)

                @pl.when(live(ci))
                def _():
                    drain_combine_store(ci, s)
            return 0

        lax.fori_loop(0, CHUNKS_PER_W // 2, pair_body, 0)

    return body(pairs, x0, x1, x2)


def _pack_pairs(values):
    u = lax.bitcast_convert_type(values, jnp.int32)
    r = ((u + 0x7FFF + ((u >> 16) & 1)) >> 16) & 0xFFFF
    rn = jnp.roll(r, -1, axis=2)
    return ((rn << 16) | r).reshape(-1)


def kernel(values, x0, x1, x2, p0, p1, p2):
    return _sc_interp(_pack_pairs(values), x0, x1, x2)


# speedup vs baseline: 1.1756x; 1.1756x over previous
import functools

import jax
import jax.numpy as jnp
from jax import lax
from jax.experimental import pallas as pl
from jax.experimental.pallas import tpu as pltpu, tpu_sc as plsc

GRID_N = 256
NUM_Q = 1000000

NC = 2
NS = 16
LANES = 16
NW = NC * NS

CHUNK = 992
CHUNKS_PER_W = 32
Q_PER_W = CHUNK * CHUNKS_PER_W

LAST_LOAD = NUM_Q - CHUNK
STRADDLE_BASE = (NUM_Q // CHUNK) * CHUNK
STRADDLE_OFF = STRADDLE_BASE - LAST_LOAD
STRADDLE_LEN = NUM_Q - STRADDLE_BASE

_OFFS = (0, GRID_N, GRID_N * GRID_N, GRID_N * GRID_N + GRID_N)


def _sc_interp(pairs, x0, x1, x2):
    mesh = plsc.VectorSubcoreMesh(
        core_axis_name="c", subcore_axis_name="s",
        num_cores=NC, num_subcores=NS)

    @functools.partial(
        pl.kernel,
        out_type=jax.ShapeDtypeStruct((NUM_Q,), jnp.float32),
        mesh=mesh,
        scratch_types=dict(
            xb=[pltpu.VMEM((CHUNK,), jnp.float32) for _ in range(3)],
            fb=[[pltpu.VMEM((CHUNK,), jnp.float32) for _ in range(3)]
                for _ in range(2)],
            idx=[[pltpu.VMEM((CHUNK,), jnp.int32) for _ in range(4)]
                 for _ in range(2)],
            cb=[[pltpu.VMEM((CHUNK,), jnp.int32) for _ in range(4)]
                for _ in range(2)],
            ob=pltpu.VMEM((CHUNK,), jnp.float32),
            sem=[pltpu.SemaphoreType.DMA for _ in range(2)],
        ),
    )
    def body(pairs_hbm, x0_hbm, x1_hbm, x2_hbm, out_hbm,
             xb, fb, idx, cb, ob, sem):
        wid = lax.axis_index("s") * NC + lax.axis_index("c")
        wbase = wid * Q_PER_W

        def compute_and_fire(ci, b):
            base = jnp.minimum(wbase + ci * CHUNK, LAST_LOAD)
            pltpu.sync_copy(x0_hbm.at[pl.ds(base, CHUNK)], xb[0])
            pltpu.sync_copy(x1_hbm.at[pl.ds(base, CHUNK)], xb[1])
            pltpu.sync_copy(x2_hbm.at[pl.ds(base, CHUNK)], xb[2])

            def compute(i, _):
                s = pl.ds(i * LANES, LANES)
                xv0 = xb[0][s]
                xv1 = xb[1][s]
                xv2 = xb[2][s]
                i0 = jnp.minimum(xv0.astype(jnp.int32), GRID_N - 2)
                i1 = jnp.minimum(xv1.astype(jnp.int32), GRID_N - 2)
                i2 = jnp.minimum(xv2.astype(jnp.int32), GRID_N - 2)
                fb[b][0][s] = xv0 - i0.astype(jnp.float32)
                fb[b][1][s] = xv1 - i1.astype(jnp.float32)
                fb[b][2][s] = xv2 - i2.astype(jnp.float32)
                flat = i0 * (GRID_N * GRID_N) + i1 * GRID_N + i2
                for k in range(4):
                    idx[b][k][s] = flat + _OFFS[k]
                return 0

            lax.fori_loop(0, CHUNK // LANES, compute, 0)
            for k in range(4):
                pltpu.async_copy(pairs_hbm.at[idx[b][k]], cb[b][k], sem[b])

        def drain_combine_store(ci, b):
            for k in range(4):
                pltpu.make_async_copy(
                    pairs_hbm.at[idx[b][k]], cb[b][k], sem[b]).wait()

            def combine(i, _):
                s = pl.ds(i * LANES, LANES)
                f0 = fb[b][0][s]
                f1 = fb[b][1][s]
                f2 = fb[b][2][s]
                w00 = cb[b][0][s]
                w01 = cb[b][1][s]
                w10 = cb[b][2][s]
                w11 = cb[b][3][s]
                c000 = lax.bitcast_convert_type(w00 << 16, jnp.float32)
                c001 = lax.bitcast_convert_type(w00 & -65536, jnp.float32)
                c010 = lax.bitcast_convert_type(w01 << 16, jnp.float32)
                c011 = lax.bitcast_convert_type(w01 & -65536, jnp.float32)
                c100 = lax.bitcast_convert_type(w10 << 16, jnp.float32)
                c101 = lax.bitcast_convert_type(w10 & -65536, jnp.float32)
                c110 = lax.bitcast_convert_type(w11 << 16, jnp.float32)
                c111 = lax.bitcast_convert_type(w11 & -65536, jnp.float32)
                v00 = c000 + f2 * (c001 - c000)
                v01 = c010 + f2 * (c011 - c010)
                v10 = c100 + f2 * (c101 - c100)
                v11 = c110 + f2 * (c111 - c110)
                v0 = v00 + f1 * (v01 - v00)
                v1 = v10 + f1 * (v11 - v10)
                ob[s] = v0 + f0 * (v1 - v0)
                return 0

            lax.fori_loop(0, CHUNK // LANES, combine, 0)
            base = wbase + ci * CHUNK

            @pl.when(base + CHUNK <= NUM_Q)
            def _():
                pltpu.sync_copy(ob, out_hbm.at[pl.ds(base, CHUNK)])

            @pl.when(base == STRADDLE_BASE)
            def _():
                pltpu.sync_copy(
                    ob.at[pl.ds(STRADDLE_OFF, STRADDLE_LEN)],
                    out_hbm.at[pl.ds(STRADDLE_BASE, STRADDLE_LEN)])

        def live(ci):
            return wbase + ci * CHUNK < NUM_Q

        @pl.when(live(0))
        def _():
            compute_and_fire(0, 0)

        def pair_body(i, _):
            for s in range(2):
                ci = 2 * i + s

                @pl.when(live(ci + 1) & (ci < CHUNKS_PER_W - 1))
                def _():
                    compute_and_fire(ci + 1, 1 - s)

                @pl.when(live(ci))
                def _():
                    drain_combine_store(ci, s)
            return 0

        lax.fori_loop(0, CHUNKS_PER_W // 2, pair_body, 0)

    return body(pairs, x0, x1, x2)


def _pack_kernel(v_ref, o_ref):
    x = v_ref[0]
    u = lax.bitcast_convert_type(x, jnp.int32)
    r = ((u + 0x7FFF + ((u >> 16) & 1)) >> 16) & 0xFFFF
    rn = pltpu.roll(r, GRID_N - 1, 1)
    o_ref[...] = ((rn << 16) | r).reshape(GRID_N * GRID_N)


def _pack_pairs(values):
    return pl.pallas_call(
        _pack_kernel,
        grid=(GRID_N,),
        in_specs=[pl.BlockSpec((1, GRID_N, GRID_N), lambda i: (i, 0, 0))],
        out_specs=pl.BlockSpec((GRID_N * GRID_N,), lambda i: (i,)),
        out_shape=jax.ShapeDtypeStruct((GRID_N ** 3,), jnp.int32),
    )(values)


def kernel(values, x0, x1, x2, p0, p1, p2):
    return _sc_interp(_pack_pairs(values), x0, x1, x2)


# speedup vs baseline: 1.7298x; 1.4715x over previous
import functools

import jax
import jax.numpy as jnp
from jax import lax
from jax.experimental import pallas as pl
from jax.experimental.pallas import tpu as pltpu, tpu_sc as plsc

GRID_N = 256
NUM_Q = 1000000

NC = 2
NS = 16
LANES = 16
NW = NC * NS

CHUNK = 992
CHUNKS_PER_W = 32
Q_PER_W = CHUNK * CHUNKS_PER_W

LAST_LOAD = NUM_Q - CHUNK
STRADDLE_BASE = (NUM_Q // CHUNK) * CHUNK
STRADDLE_OFF = STRADDLE_BASE - LAST_LOAD
STRADDLE_LEN = NUM_Q - STRADDLE_BASE

_OFFS = (0, GRID_N, GRID_N * GRID_N, GRID_N * GRID_N + GRID_N)


def _sc_interp(pairs, x0, x1, x2):
    mesh = plsc.VectorSubcoreMesh(
        core_axis_name="c", subcore_axis_name="s",
        num_cores=NC, num_subcores=NS)

    @functools.partial(
        pl.kernel,
        out_type=jax.ShapeDtypeStruct((NUM_Q,), jnp.float32),
        mesh=mesh,
        scratch_types=dict(
            xb=[pltpu.VMEM((CHUNK,), jnp.float32) for _ in range(3)],
            fb=[[pltpu.VMEM((CHUNK,), jnp.float32) for _ in range(3)]
                for _ in range(2)],
            idx=[[pltpu.VMEM((CHUNK,), jnp.int32) for _ in range(4)]
                 for _ in range(2)],
            cb=[[pltpu.VMEM((CHUNK,), jnp.int32) for _ in range(4)]
                for _ in range(2)],
            ob=pltpu.VMEM((CHUNK,), jnp.float32),
            sem=[pltpu.SemaphoreType.DMA for _ in range(2)],
        ),
    )
    def body(pairs_hbm, x0_hbm, x1_hbm, x2_hbm, out_hbm,
             xb, fb, idx, cb, ob, sem):
        wid = lax.axis_index("s") * NC + lax.axis_index("c")
        wbase = wid * Q_PER_W

        def compute_and_fire(ci, b):
            base = jnp.minimum(wbase + ci * CHUNK, LAST_LOAD)
            pltpu.sync_copy(x0_hbm.at[pl.ds(base, CHUNK)], xb[0])
            pltpu.sync_copy(x1_hbm.at[pl.ds(base, CHUNK)], xb[1])
            pltpu.sync_copy(x2_hbm.at[pl.ds(base, CHUNK)], xb[2])

            def compute(i, _):
                s = pl.ds(i * LANES, LANES)
                xv0 = xb[0][s]
                xv1 = xb[1][s]
                xv2 = xb[2][s]
                i0 = jnp.minimum(xv0.astype(jnp.int32), GRID_N - 2)
                i1 = jnp.minimum(xv1.astype(jnp.int32), GRID_N - 2)
                i2 = jnp.minimum(xv2.astype(jnp.int32), GRID_N - 2)
                fb[b][0][s] = xv0 - i0.astype(jnp.float32)
                fb[b][1][s] = xv1 - i1.astype(jnp.float32)
                fb[b][2][s] = xv2 - i2.astype(jnp.float32)
                flat = i0 * (GRID_N * GRID_N) + i1 * GRID_N + i2
                for k in range(4):
                    idx[b][k][s] = flat + _OFFS[k]
                return 0

            lax.fori_loop(0, CHUNK // LANES, compute, 0)
            for k in range(4):
                pltpu.async_copy(pairs_hbm.at[idx[b][k]], cb[b][k], sem[b])

        def drain_combine_store(ci, b):
            for k in range(4):
                pltpu.make_async_copy(
                    pairs_hbm.at[idx[b][k]], cb[b][k], sem[b]).wait()

            def combine(i, _):
                s = pl.ds(i * LANES, LANES)
                f0 = fb[b][0][s]
                f1 = fb[b][1][s]
                f2 = fb[b][2][s]
                w00 = cb[b][0][s]
                w01 = cb[b][1][s]
                w10 = cb[b][2][s]
                w11 = cb[b][3][s]
                c000 = lax.bitcast_convert_type(w00 << 16, jnp.float32)
                c001 = lax.bitcast_convert_type(w00 & -65536, jnp.float32)
                c010 = lax.bitcast_convert_type(w01 << 16, jnp.float32)
                c011 = lax.bitcast_convert_type(w01 & -65536, jnp.float32)
                c100 = lax.bitcast_convert_type(w10 << 16, jnp.float32)
                c101 = lax.bitcast_convert_type(w10 & -65536, jnp.float32)
                c110 = lax.bitcast_convert_type(w11 << 16, jnp.float32)
                c111 = lax.bitcast_convert_type(w11 & -65536, jnp.float32)
                v00 = c000 + f2 * (c001 - c000)
                v01 = c010 + f2 * (c011 - c010)
                v10 = c100 + f2 * (c101 - c100)
                v11 = c110 + f2 * (c111 - c110)
                v0 = v00 + f1 * (v01 - v00)
                v1 = v10 + f1 * (v11 - v10)
                ob[s] = v0 + f0 * (v1 - v0)
                return 0

            lax.fori_loop(0, CHUNK // LANES, combine, 0)
            base = wbase + ci * CHUNK

            @pl.when(base + CHUNK <= NUM_Q)
            def _():
                pltpu.sync_copy(ob, out_hbm.at[pl.ds(base, CHUNK)])

            @pl.when(base == STRADDLE_BASE)
            def _():
                pltpu.sync_copy(
                    ob.at[pl.ds(STRADDLE_OFF, STRADDLE_LEN)],
                    out_hbm.at[pl.ds(STRADDLE_BASE, STRADDLE_LEN)])

        def live(ci):
            return wbase + ci * CHUNK < NUM_Q

        @pl.when(live(0))
        def _():
            compute_and_fire(0, 0)

        def pair_body(i, _):
            for s in range(2):
                ci = 2 * i + s

                @pl.when(live(ci + 1) & (ci < CHUNKS_PER_W - 1))
                def _():
                    compute_and_fire(ci + 1, 1 - s)

                @pl.when(live(ci))
                def _():
                    drain_combine_store(ci, s)
            return 0

        lax.fori_loop(0, CHUNKS_PER_W // 2, pair_body, 0)

    return body(pairs, x0, x1, x2)


_PACK_B0 = 8


def _pack_kernel(v_ref, o_ref):
    for s in range(_PACK_B0):
        x = v_ref[s]
        u = lax.bitcast_convert_type(x, jnp.int32)
        r = ((u + 0x7FFF + ((u >> 16) & 1)) >> 16) & 0xFFFF
        rn = pltpu.roll(r, GRID_N - 1, 1)
        w = ((rn << 16) | r).reshape(GRID_N * GRID_N)
        o_ref[pl.ds(s * GRID_N * GRID_N, GRID_N * GRID_N)] = w


def _pack_pairs(values):
    return pl.pallas_call(
        _pack_kernel,
        grid=(GRID_N // _PACK_B0,),
        in_specs=[pl.BlockSpec((_PACK_B0, GRID_N, GRID_N), lambda i: (i, 0, 0))],
        out_specs=pl.BlockSpec((_PACK_B0 * GRID_N * GRID_N,), lambda i: (i,)),
        out_shape=jax.ShapeDtypeStruct((GRID_N ** 3,), jnp.int32),
    )(values)


def kernel(values, x0, x1, x2, p0, p1, p2):
    return _sc_interp(_pack_pairs(values), x0, x1, x2)


# speedup vs baseline: 1.7867x; 1.0329x over previous
import functools

import jax
import jax.numpy as jnp
from jax import lax
from jax.experimental import pallas as pl
from jax.experimental.pallas import tpu as pltpu, tpu_sc as plsc

GRID_N = 256
NUM_Q = 1000000

NC = 2
NS = 16
LANES = 16
NW = NC * NS

CHUNK = 1984
CHUNKS_PER_W = 16
Q_PER_W = CHUNK * CHUNKS_PER_W

LAST_LOAD = NUM_Q - CHUNK
STRADDLE_BASE = (NUM_Q // CHUNK) * CHUNK
STRADDLE_OFF = STRADDLE_BASE - LAST_LOAD
STRADDLE_LEN = NUM_Q - STRADDLE_BASE

_OFFS = (0, GRID_N, GRID_N * GRID_N, GRID_N * GRID_N + GRID_N)


def _sc_interp(pairs, x0, x1, x2):
    mesh = plsc.VectorSubcoreMesh(
        core_axis_name="c", subcore_axis_name="s",
        num_cores=NC, num_subcores=NS)

    @functools.partial(
        pl.kernel,
        out_type=jax.ShapeDtypeStruct((NUM_Q,), jnp.float32),
        mesh=mesh,
        scratch_types=dict(
            xb=[pltpu.VMEM((CHUNK,), jnp.float32) for _ in range(3)],
            fb=[[pltpu.VMEM((CHUNK,), jnp.float32) for _ in range(3)]
                for _ in range(2)],
            idx=[[pltpu.VMEM((CHUNK,), jnp.int32) for _ in range(4)]
                 for _ in range(2)],
            cb=[[pltpu.VMEM((CHUNK,), jnp.int32) for _ in range(4)]
                for _ in range(2)],
            ob=pltpu.VMEM((CHUNK,), jnp.float32),
            sem=[pltpu.SemaphoreType.DMA for _ in range(2)],
        ),
    )
    def body(pairs_hbm, x0_hbm, x1_hbm, x2_hbm, out_hbm,
             xb, fb, idx, cb, ob, sem):
        wid = lax.axis_index("s") * NC + lax.axis_index("c")
        wbase = wid * Q_PER_W

        def compute_and_fire(ci, b):
            base = jnp.minimum(wbase + ci * CHUNK, LAST_LOAD)
            pltpu.sync_copy(x0_hbm.at[pl.ds(base, CHUNK)], xb[0])
            pltpu.sync_copy(x1_hbm.at[pl.ds(base, CHUNK)], xb[1])
            pltpu.sync_copy(x2_hbm.at[pl.ds(base, CHUNK)], xb[2])

            def compute(i, _):
                s = pl.ds(i * LANES, LANES)
                xv0 = xb[0][s]
                xv1 = xb[1][s]
                xv2 = xb[2][s]
                i0 = jnp.minimum(xv0.astype(jnp.int32), GRID_N - 2)
                i1 = jnp.minimum(xv1.astype(jnp.int32), GRID_N - 2)
                i2 = jnp.minimum(xv2.astype(jnp.int32), GRID_N - 2)
                fb[b][0][s] = xv0 - i0.astype(jnp.float32)
                fb[b][1][s] = xv1 - i1.astype(jnp.float32)
                fb[b][2][s] = xv2 - i2.astype(jnp.float32)
                flat = i0 * (GRID_N * GRID_N) + i1 * GRID_N + i2
                for k in range(4):
                    idx[b][k][s] = flat + _OFFS[k]
                return 0

            lax.fori_loop(0, CHUNK // LANES, compute, 0)
            for k in range(4):
                pltpu.async_copy(pairs_hbm.at[idx[b][k]], cb[b][k], sem[b])

        def drain_combine_store(ci, b):
            for k in range(4):
                pltpu.make_async_copy(
                    pairs_hbm.at[idx[b][k]], cb[b][k], sem[b]).wait()

            def combine(i, _):
                s = pl.ds(i * LANES, LANES)
                f0 = fb[b][0][s]
                f1 = fb[b][1][s]
                f2 = fb[b][2][s]
                w00 = cb[b][0][s]
                w01 = cb[b][1][s]
                w10 = cb[b][2][s]
                w11 = cb[b][3][s]
                c000 = lax.bitcast_convert_type(w00 << 16, jnp.float32)
                c001 = lax.bitcast_convert_type(w00 & -65536, jnp.float32)
                c010 = lax.bitcast_convert_type(w01 << 16, jnp.float32)
                c011 = lax.bitcast_convert_type(w01 & -65536, jnp.float32)
                c100 = lax.bitcast_convert_type(w10 << 16, jnp.float32)
                c101 = lax.bitcast_convert_type(w10 & -65536, jnp.float32)
                c110 = lax.bitcast_convert_type(w11 << 16, jnp.float32)
                c111 = lax.bitcast_convert_type(w11 & -65536, jnp.float32)
                v00 = c000 + f2 * (c001 - c000)
                v01 = c010 + f2 * (c011 - c010)
                v10 = c100 + f2 * (c101 - c100)
                v11 = c110 + f2 * (c111 - c110)
                v0 = v00 + f1 * (v01 - v00)
                v1 = v10 + f1 * (v11 - v10)
                ob[s] = v0 + f0 * (v1 - v0)
                return 0

            lax.fori_loop(0, CHUNK // LANES, combine, 0)
            base = wbase + ci * CHUNK

            @pl.when(base + CHUNK <= NUM_Q)
            def _():
                pltpu.sync_copy(ob, out_hbm.at[pl.ds(base, CHUNK)])

            @pl.when(base == STRADDLE_BASE)
            def _():
                pltpu.sync_copy(
                    ob.at[pl.ds(STRADDLE_OFF, STRADDLE_LEN)],
                    out_hbm.at[pl.ds(STRADDLE_BASE, STRADDLE_LEN)])

        def live(ci):
            return wbase + ci * CHUNK < NUM_Q

        @pl.when(live(0))
        def _():
            compute_and_fire(0, 0)

        def pair_body(i, _):
            for s in range(2):
                ci = 2 * i + s

                @pl.when(live(ci + 1) & (ci < CHUNKS_PER_W - 1))
                def _():
                    compute_and_fire(ci + 1, 1 - s)

                @pl.when(live(ci))
                def _():
                    drain_combine_store(ci, s)
            return 0

        lax.fori_loop(0, CHUNKS_PER_W // 2, pair_body, 0)

    return body(pairs, x0, x1, x2)


_PACK_B0 = 8


def _pack_kernel(v_ref, o_ref):
    for s in range(_PACK_B0):
        x = v_ref[s]
        u = lax.bitcast_convert_type(x, jnp.int32)
        r = ((u + 0x7FFF + ((u >> 16) & 1)) >> 16) & 0xFFFF
        rn = pltpu.roll(r, GRID_N - 1, 1)
        w = ((rn << 16) | r).reshape(GRID_N * GRID_N)
        o_ref[pl.ds(s * GRID_N * GRID_N, GRID_N * GRID_N)] = w


def _pack_pairs(values):
    return pl.pallas_call(
        _pack_kernel,
        grid=(GRID_N // _PACK_B0,),
        in_specs=[pl.BlockSpec((_PACK_B0, GRID_N, GRID_N), lambda i: (i, 0, 0))],
        out_specs=pl.BlockSpec((_PACK_B0 * GRID_N * GRID_N,), lambda i: (i,)),
        out_shape=jax.ShapeDtypeStruct((GRID_N ** 3,), jnp.int32),
    )(values)


def kernel(values, x0, x1, x2, p0, p1, p2):
    return _sc_interp(_pack_pairs(values), x0, x1, x2)


# speedup vs baseline: 1.8006x; 1.0078x over previous
import functools

import jax
import jax.numpy as jnp
from jax import lax
from jax.experimental import pallas as pl
from jax.experimental.pallas import tpu as pltpu, tpu_sc as plsc

GRID_N = 256
NUM_Q = 1000000

NC = 2
NS = 16
LANES = 16
NW = NC * NS

CHUNK = 3968
CHUNKS_PER_W = 8
Q_PER_W = CHUNK * CHUNKS_PER_W

LAST_LOAD = NUM_Q - CHUNK
STRADDLE_BASE = (NUM_Q // CHUNK) * CHUNK
STRADDLE_OFF = STRADDLE_BASE - LAST_LOAD
STRADDLE_LEN = NUM_Q - STRADDLE_BASE

_OFFS = (0, GRID_N, GRID_N * GRID_N, GRID_N * GRID_N + GRID_N)


def _sc_interp(pairs, x0, x1, x2):
    mesh = plsc.VectorSubcoreMesh(
        core_axis_name="c", subcore_axis_name="s",
        num_cores=NC, num_subcores=NS)

    @functools.partial(
        pl.kernel,
        out_type=jax.ShapeDtypeStruct((NUM_Q,), jnp.float32),
        mesh=mesh,
        scratch_types=dict(
            xb=[pltpu.VMEM((CHUNK,), jnp.float32) for _ in range(3)],
            fb=[[pltpu.VMEM((CHUNK,), jnp.float32) for _ in range(3)]
                for _ in range(2)],
            idx=[[pltpu.VMEM((CHUNK,), jnp.int32) for _ in range(4)]
                 for _ in range(2)],
            cb=[[pltpu.VMEM((CHUNK,), jnp.int32) for _ in range(4)]
                for _ in range(2)],
            ob=pltpu.VMEM((CHUNK,), jnp.float32),
            sem=[pltpu.SemaphoreType.DMA for _ in range(2)],
        ),
    )
    def body(pairs_hbm, x0_hbm, x1_hbm, x2_hbm, out_hbm,
             xb, fb, idx, cb, ob, sem):
        wid = lax.axis_index("s") * NC + lax.axis_index("c")
        wbase = wid * Q_PER_W

        def compute_and_fire(ci, b):
            base = jnp.minimum(wbase + ci * CHUNK, LAST_LOAD)
            pltpu.sync_copy(x0_hbm.at[pl.ds(base, CHUNK)], xb[0])
            pltpu.sync_copy(x1_hbm.at[pl.ds(base, CHUNK)], xb[1])
            pltpu.sync_copy(x2_hbm.at[pl.ds(base, CHUNK)], xb[2])

            def compute(i, _):
                s = pl.ds(i * LANES, LANES)
                xv0 = xb[0][s]
                xv1 = xb[1][s]
                xv2 = xb[2][s]
                i0 = jnp.minimum(xv0.astype(jnp.int32), GRID_N - 2)
                i1 = jnp.minimum(xv1.astype(jnp.int32), GRID_N - 2)
                i2 = jnp.minimum(xv2.astype(jnp.int32), GRID_N - 2)
                fb[b][0][s] = xv0 - i0.astype(jnp.float32)
                fb[b][1][s] = xv1 - i1.astype(jnp.float32)
                fb[b][2][s] = xv2 - i2.astype(jnp.float32)
                flat = i0 * (GRID_N * GRID_N) + i1 * GRID_N + i2
                for k in range(4):
                    idx[b][k][s] = flat + _OFFS[k]
                return 0

            lax.fori_loop(0, CHUNK // LANES, compute, 0)
            for k in range(4):
                pltpu.async_copy(pairs_hbm.at[idx[b][k]], cb[b][k], sem[b])

        def drain_combine_store(ci, b):
            for k in range(4):
                pltpu.make_async_copy(
                    pairs_hbm.at[idx[b][k]], cb[b][k], sem[b]).wait()

            def combine(i, _):
                s = pl.ds(i * LANES, LANES)
                f0 = fb[b][0][s]
                f1 = fb[b][1][s]
                f2 = fb[b][2][s]
                w00 = cb[b][0][s]
                w01 = cb[b][1][s]
                w10 = cb[b][2][s]
                w11 = cb[b][3][s]
                c000 = lax.bitcast_convert_type(w00 << 16, jnp.float32)
                c001 = lax.bitcast_convert_type(w00 & -65536, jnp.float32)
                c010 = lax.bitcast_convert_type(w01 << 16, jnp.float32)
                c011 = lax.bitcast_convert_type(w01 & -65536, jnp.float32)
                c100 = lax.bitcast_convert_type(w10 << 16, jnp.float32)
                c101 = lax.bitcast_convert_type(w10 & -65536, jnp.float32)
                c110 = lax.bitcast_convert_type(w11 << 16, jnp.float32)
                c111 = lax.bitcast_convert_type(w11 & -65536, jnp.float32)
                v00 = c000 + f2 * (c001 - c000)
                v01 = c010 + f2 * (c011 - c010)
                v10 = c100 + f2 * (c101 - c100)
                v11 = c110 + f2 * (c111 - c110)
                v0 = v00 + f1 * (v01 - v00)
                v1 = v10 + f1 * (v11 - v10)
                ob[s] = v0 + f0 * (v1 - v0)
                return 0

            lax.fori_loop(0, CHUNK // LANES, combine, 0)
            base = wbase + ci * CHUNK

            @pl.when(base + CHUNK <= NUM_Q)
            def _():
                pltpu.sync_copy(ob, out_hbm.at[pl.ds(base, CHUNK)])

            @pl.when(base == STRADDLE_BASE)
            def _():
                pltpu.sync_copy(
                    ob.at[pl.ds(STRADDLE_OFF, STRADDLE_LEN)],
                    out_hbm.at[pl.ds(STRADDLE_BASE, STRADDLE_LEN)])

        def live(ci):
            return wbase + ci * CHUNK < NUM_Q

        @pl.when(live(0))
        def _():
            compute_and_fire(0, 0)

        def pair_body(i, _):
            for s in range(2):
                ci = 2 * i + s

                @pl.when(live(ci + 1) & (ci < CHUNKS_PER_W - 1))
                def _():
                    compute_and_fire(ci + 1, 1 - s)

                @pl.when(live(ci))
                def _():
                    drain_combine_store(ci, s)
            return 0

        lax.fori_loop(0, CHUNKS_PER_W // 2, pair_body, 0)

    return body(pairs, x0, x1, x2)


_PACK_B0 = 8


def _pack_kernel(v_ref, o_ref):
    for s in range(_PACK_B0):
        x = v_ref[s]
        u = lax.bitcast_convert_type(x, jnp.int32)
        r = ((u + 0x7FFF + ((u >> 16) & 1)) >> 16) & 0xFFFF
        rn = pltpu.roll(r, GRID_N - 1, 1)
        w = ((rn << 16) | r).reshape(GRID_N * GRID_N)
        o_ref[pl.ds(s * GRID_N * GRID_N, GRID_N * GRID_N)] = w


def _pack_pairs(values):
    return pl.pallas_call(
        _pack_kernel,
        grid=(GRID_N // _PACK_B0,),
        in_specs=[pl.BlockSpec((_PACK_B0, GRID_N, GRID_N), lambda i: (i, 0, 0))],
        out_specs=pl.BlockSpec((_PACK_B0 * GRID_N * GRID_N,), lambda i: (i,)),
        out_shape=jax.ShapeDtypeStruct((GRID_N ** 3,), jnp.int32),
    )(values)


def kernel(values, x0, x1, x2, p0, p1, p2):
    return _sc_interp(_pack_pairs(values), x0, x1, x2)


# speedup vs baseline: 1.8649x; 1.0357x over previous
import functools

import jax
import jax.numpy as jnp
from jax import lax
from jax.experimental import pallas as pl
from jax.experimental.pallas import tpu as pltpu, tpu_sc as plsc

GRID_N = 256
NUM_Q = 1000000

NC = 2
NS = 16
LANES = 16
NW = NC * NS

CHUNK = 3968
CHUNKS_PER_W = 8
Q_PER_W = CHUNK * CHUNKS_PER_W

LAST_LOAD = NUM_Q - CHUNK
STRADDLE_BASE = (NUM_Q // CHUNK) * CHUNK
STRADDLE_OFF = STRADDLE_BASE - LAST_LOAD
STRADDLE_LEN = NUM_Q - STRADDLE_BASE

_OFFS = (0, GRID_N, GRID_N * GRID_N, GRID_N * GRID_N + GRID_N)


def _sc_interp(pairs, x0, x1, x2):
    mesh = plsc.VectorSubcoreMesh(
        core_axis_name="c", subcore_axis_name="s",
        num_cores=NC, num_subcores=NS)

    @functools.partial(
        pl.kernel,
        out_type=jax.ShapeDtypeStruct((NUM_Q,), jnp.float32),
        mesh=mesh,
        scratch_types=dict(
            xb=[pltpu.VMEM((CHUNK,), jnp.float32) for _ in range(3)],
            fb=[[pltpu.VMEM((CHUNK,), jnp.float32) for _ in range(3)]
                for _ in range(2)],
            idx=[[pltpu.VMEM((CHUNK,), jnp.int32) for _ in range(4)]
                 for _ in range(2)],
            cb=[[pltpu.VMEM((CHUNK,), jnp.int32) for _ in range(4)]
                for _ in range(2)],
            ob=pltpu.VMEM((CHUNK,), jnp.float32),
            sem=[pltpu.SemaphoreType.DMA for _ in range(2)],
        ),
    )
    def body(pairs_hbm, x0_hbm, x1_hbm, x2_hbm, out_hbm,
             xb, fb, idx, cb, ob, sem):
        wid = lax.axis_index("s") * NC + lax.axis_index("c")
        wbase = wid * Q_PER_W

        def compute_and_fire(ci, b):
            base = jnp.minimum(wbase + ci * CHUNK, LAST_LOAD)
            pltpu.sync_copy(x0_hbm.at[pl.ds(base, CHUNK)], xb[0])
            pltpu.sync_copy(x1_hbm.at[pl.ds(base, CHUNK)], xb[1])
            pltpu.sync_copy(x2_hbm.at[pl.ds(base, CHUNK)], xb[2])

            def compute(i, _):
                s = pl.ds(i * LANES, LANES)
                xv0 = xb[0][s]
                xv1 = xb[1][s]
                xv2 = xb[2][s]
                i0 = jnp.minimum(xv0.astype(jnp.int32), GRID_N - 2)
                i1 = jnp.minimum(xv1.astype(jnp.int32), GRID_N - 2)
                i2 = jnp.minimum(xv2.astype(jnp.int32), GRID_N - 2)
                fb[b][0][s] = xv0 - i0.astype(jnp.float32)
                fb[b][1][s] = xv1 - i1.astype(jnp.float32)
                fb[b][2][s] = xv2 - i2.astype(jnp.float32)
                flat = i0 * (GRID_N * GRID_N) + i1 * GRID_N + i2
                for k in range(4):
                    idx[b][k][s] = flat + _OFFS[k]
                return 0

            lax.fori_loop(0, CHUNK // LANES, compute, 0)
            for k in range(4):
                pltpu.async_copy(pairs_hbm.at[idx[b][k]], cb[b][k], sem[b])

        def drain_combine_store(ci, b):
            for k in range(4):
                pltpu.make_async_copy(
                    pairs_hbm.at[idx[b][k]], cb[b][k], sem[b]).wait()

            def combine(i, _):
                s = pl.ds(i * LANES, LANES)
                f0 = fb[b][0][s]
                f1 = fb[b][1][s]
                f2 = fb[b][2][s]
                w00 = cb[b][0][s]
                w01 = cb[b][1][s]
                w10 = cb[b][2][s]
                w11 = cb[b][3][s]
                c000 = lax.bitcast_convert_type(w00 << 16, jnp.float32)
                c001 = lax.bitcast_convert_type(w00 & -65536, jnp.float32)
                c010 = lax.bitcast_convert_type(w01 << 16, jnp.float32)
                c011 = lax.bitcast_convert_type(w01 & -65536, jnp.float32)
                c100 = lax.bitcast_convert_type(w10 << 16, jnp.float32)
                c101 = lax.bitcast_convert_type(w10 & -65536, jnp.float32)
                c110 = lax.bitcast_convert_type(w11 << 16, jnp.float32)
                c111 = lax.bitcast_convert_type(w11 & -65536, jnp.float32)
                v00 = c000 + f2 * (c001 - c000)
                v01 = c010 + f2 * (c011 - c010)
                v10 = c100 + f2 * (c101 - c100)
                v11 = c110 + f2 * (c111 - c110)
                v0 = v00 + f1 * (v01 - v00)
                v1 = v10 + f1 * (v11 - v10)
                ob[s] = v0 + f0 * (v1 - v0)
                return 0

            lax.fori_loop(0, CHUNK // LANES, combine, 0)
            base = wbase + ci * CHUNK

            @pl.when(base + CHUNK <= NUM_Q)
            def _():
                pltpu.sync_copy(ob, out_hbm.at[pl.ds(base, CHUNK)])

            @pl.when(base == STRADDLE_BASE)
            def _():
                pltpu.sync_copy(
                    ob.at[pl.ds(STRADDLE_OFF, STRADDLE_LEN)],
                    out_hbm.at[pl.ds(STRADDLE_BASE, STRADDLE_LEN)])

        def live(ci):
            return wbase + ci * CHUNK < NUM_Q

        @pl.when(live(0))
        def _():
            compute_and_fire(0, 0)

        def pair_body(i, _):
            for s in range(2):
                ci = 2 * i + s

                @pl.when(live(ci + 1) & (ci < CHUNKS_PER_W - 1))
                def _():
                    compute_and_fire(ci + 1, 1 - s)

                @pl.when(live(ci))
                def _():
                    drain_combine_store(ci, s)
            return 0

        lax.fori_loop(0, CHUNKS_PER_W // 2, pair_body, 0)

    return body(pairs, x0, x1, x2)


_PACK_B0 = 16


def _pack_kernel(v_ref, o_ref):
    for s in range(_PACK_B0):
        x = v_ref[s]
        u = lax.bitcast_convert_type(x, jnp.int32)
        r = ((u + 0x7FFF + ((u >> 16) & 1)) >> 16) & 0xFFFF
        rn = pltpu.roll(r, GRID_N - 1, 1)
        w = ((rn << 16) | r).reshape(GRID_N * GRID_N)
        o_ref[pl.ds(s * GRID_N * GRID_N, GRID_N * GRID_N)] = w


def _pack_pairs(values):
    return pl.pallas_call(
        _pack_kernel,
        grid=(GRID_N // _PACK_B0,),
        in_specs=[pl.BlockSpec((_PACK_B0, GRID_N, GRID_N), lambda i: (i, 0, 0))],
        out_specs=pl.BlockSpec((_PACK_B0 * GRID_N * GRID_N,), lambda i: (i,)),
        out_shape=jax.ShapeDtypeStruct((GRID_N ** 3,), jnp.int32),
    )(values)


def kernel(values, x0, x1, x2, p0, p1, p2):
    return _sc_interp(_pack_pairs(values), x0, x1, x2)


# speedup vs baseline: 1.8709x; 1.0032x over previous
import functools

import jax
import jax.numpy as jnp
from jax import lax
from jax.experimental import pallas as pl
from jax.experimental.pallas import tpu as pltpu, tpu_sc as plsc

GRID_N = 256
NUM_Q = 1000000

NC = 2
NS = 16
LANES = 16
NW = NC * NS

CHUNK = 3968
CHUNKS_PER_W = 8
Q_PER_W = CHUNK * CHUNKS_PER_W

LAST_LOAD = NUM_Q - CHUNK
STRADDLE_BASE = (NUM_Q // CHUNK) * CHUNK
STRADDLE_OFF = STRADDLE_BASE - LAST_LOAD
STRADDLE_LEN = NUM_Q - STRADDLE_BASE

_OFFS = (0, GRID_N, GRID_N * GRID_N, GRID_N * GRID_N + GRID_N)


def _sc_interp(pairs, x0, x1, x2):
    mesh = plsc.VectorSubcoreMesh(
        core_axis_name="c", subcore_axis_name="s",
        num_cores=NC, num_subcores=NS)

    @functools.partial(
        pl.kernel,
        out_type=jax.ShapeDtypeStruct((NUM_Q,), jnp.float32),
        mesh=mesh,
        scratch_types=dict(
            xb=[pltpu.VMEM((CHUNK,), jnp.float32) for _ in range(3)],
            fb=[[pltpu.VMEM((CHUNK,), jnp.float32) for _ in range(3)]
                for _ in range(2)],
            idx=[[pltpu.VMEM((CHUNK,), jnp.int32) for _ in range(4)]
                 for _ in range(2)],
            cb=[[pltpu.VMEM((CHUNK,), jnp.int32) for _ in range(4)]
                for _ in range(2)],
            ob=pltpu.VMEM((CHUNK,), jnp.float32),
            sem=[pltpu.SemaphoreType.DMA for _ in range(2)],
        ),
    )
    def body(pairs_hbm, x0_hbm, x1_hbm, x2_hbm, out_hbm,
             xb, fb, idx, cb, ob, sem):
        wid = lax.axis_index("s") * NC + lax.axis_index("c")
        wbase = wid * Q_PER_W

        def compute_and_fire(ci, b):
            base = jnp.minimum(wbase + ci * CHUNK, LAST_LOAD)
            pltpu.sync_copy(x0_hbm.at[pl.ds(base, CHUNK)], xb[0])
            pltpu.sync_copy(x1_hbm.at[pl.ds(base, CHUNK)], xb[1])
            pltpu.sync_copy(x2_hbm.at[pl.ds(base, CHUNK)], xb[2])

            def compute(i, _):
                s = pl.ds(i * LANES, LANES)
                xv0 = xb[0][s]
                xv1 = xb[1][s]
                xv2 = xb[2][s]
                i0 = jnp.minimum(xv0.astype(jnp.int32), GRID_N - 2)
                i1 = jnp.minimum(xv1.astype(jnp.int32), GRID_N - 2)
                i2 = jnp.minimum(xv2.astype(jnp.int32), GRID_N - 2)
                fb[b][0][s] = xv0 - i0.astype(jnp.float32)
                fb[b][1][s] = xv1 - i1.astype(jnp.float32)
                fb[b][2][s] = xv2 - i2.astype(jnp.float32)
                flat = i0 * (GRID_N * GRID_N) + i1 * GRID_N + i2
                for k in range(4):
                    idx[b][k][s] = flat + _OFFS[k]
                return 0

            lax.fori_loop(0, CHUNK // LANES, compute, 0)
            for k in range(4):
                pltpu.async_copy(pairs_hbm.at[idx[b][k]], cb[b][k], sem[b])

        def drain_combine_store(ci, b):
            for k in range(4):
                pltpu.make_async_copy(
                    pairs_hbm.at[idx[b][k]], cb[b][k], sem[b]).wait()

            def combine(i, _):
                s = pl.ds(i * LANES, LANES)
                f0 = fb[b][0][s]
                f1 = fb[b][1][s]
                f2 = fb[b][2][s]
                w00 = cb[b][0][s]
                w01 = cb[b][1][s]
                w10 = cb[b][2][s]
                w11 = cb[b][3][s]
                c000 = lax.bitcast_convert_type(w00 << 16, jnp.float32)
                c001 = lax.bitcast_convert_type(w00 & -65536, jnp.float32)
                c010 = lax.bitcast_convert_type(w01 << 16, jnp.float32)
                c011 = lax.bitcast_convert_type(w01 & -65536, jnp.float32)
                c100 = lax.bitcast_convert_type(w10 << 16, jnp.float32)
                c101 = lax.bitcast_convert_type(w10 & -65536, jnp.float32)
                c110 = lax.bitcast_convert_type(w11 << 16, jnp.float32)
                c111 = lax.bitcast_convert_type(w11 & -65536, jnp.float32)
                v00 = c000 + f2 * (c001 - c000)
                v01 = c010 + f2 * (c011 - c010)
                v10 = c100 + f2 * (c101 - c100)
                v11 = c110 + f2 * (c111 - c110)
                v0 = v00 + f1 * (v01 - v00)
                v1 = v10 + f1 * (v11 - v10)
                ob[s] = v0 + f0 * (v1 - v0)
                return 0

            lax.fori_loop(0, CHUNK // LANES, combine, 0)
            base = wbase + ci * CHUNK

            @pl.when(base + CHUNK <= NUM_Q)
            def _():
                pltpu.sync_copy(ob, out_hbm.at[pl.ds(base, CHUNK)])

            @pl.when(base == STRADDLE_BASE)
            def _():
                pltpu.sync_copy(
                    ob.at[pl.ds(STRADDLE_OFF, STRADDLE_LEN)],
                    out_hbm.at[pl.ds(STRADDLE_BASE, STRADDLE_LEN)])

        def live(ci):
            return wbase + ci * CHUNK < NUM_Q

        @pl.when(live(0))
        def _():
            compute_and_fire(0, 0)

        def pair_body(i, _):
            for s in range(2):
                ci = 2 * i + s

                @pl.when(live(ci + 1) & (ci < CHUNKS_PER_W - 1))
                def _():
                    compute_and_fire(ci + 1, 1 - s)

                @pl.when(live(ci))
                def _():
                    drain_combine_store(ci, s)
            return 0

        lax.fori_loop(0, CHUNKS_PER_W // 2, pair_body, 0)

    return body(pairs, x0, x1, x2)


_PACK_B0 = 32


def _pack_kernel(v_ref, o_ref):
    for s in range(_PACK_B0):
        x = v_ref[s]
        u = lax.bitcast_convert_type(x, jnp.int32)
        r = ((u + 0x7FFF + ((u >> 16) & 1)) >> 16) & 0xFFFF
        rn = pltpu.roll(r, GRID_N - 1, 1)
        w = ((rn << 16) | r).reshape(GRID_N * GRID_N)
        o_ref[pl.ds(s * GRID_N * GRID_N, GRID_N * GRID_N)] = w


def _pack_pairs(values):
    return pl.pallas_call(
        _pack_kernel,
        grid=(GRID_N // _PACK_B0,),
        in_specs=[pl.BlockSpec((_PACK_B0, GRID_N, GRID_N), lambda i: (i, 0, 0))],
        out_specs=pl.BlockSpec((_PACK_B0 * GRID_N * GRID_N,), lambda i: (i,)),
        out_shape=jax.ShapeDtypeStruct((GRID_N ** 3,), jnp.int32),
    )(values)


def kernel(values, x0, x1, x2, p0, p1, p2):
    return _sc_interp(_pack_pairs(values), x0, x1, x2)
